# baseline (speedup 1.0000x reference)
import functools

import jax
import jax.numpy as jnp
from jax import lax
from jax.experimental import pallas as pl
from jax.experimental.pallas import tpu as pltpu
from jax.experimental.pallas import tpu_sc as plsc

_NUM_CLASSES = 21
_N = 1048576
_NW = 32
_ROWS_PER_TILE = _N // _NW
_CHUNK = 1024
_CW = _CHUNK * _NUM_CLASSES
_NCHUNKS = _ROWS_PER_TILE // _CHUNK
_GROUPS = _CHUNK // 16
_HIST_PAD = 448


def _argmax16(chunk_ref, row_base):
    nodes = []
    for c in range(_NUM_CLASSES):
        v = plsc.load_gather(chunk_ref, [row_base + c])
        nodes.append((v, jnp.full((16,), c, jnp.int32)))
    while len(nodes) > 1:
        nxt = []
        for i in range(0, len(nodes) - 1, 2):
            (va, ia), (vb, ib) = nodes[i], nodes[i + 1]
            m = vb > va
            nxt.append((jnp.where(m, vb, va), jnp.where(m, ib, ia)))
        if len(nodes) % 2:
            nxt.append(nodes[-1])
        nodes = nxt
    return nodes[0][1]


def _cm_body(out_hbm, tgt_hbm, part_hbm, chunk0, chunk1, tgt0, tgt1,
             hist_v, res_v, shf, sht, semh0, semh1, semx0, semx1):
    s_id = lax.axis_index("s")
    wid = lax.axis_index("c") * 16 + s_id
    lanes = lax.broadcasted_iota(jnp.int32, (16,), 0)
    zeros_f = jnp.zeros((16,), jnp.float32)
    ones_f = jnp.ones((16,), jnp.float32)
    chunks = (chunk0, chunk1)
    tgts = (tgt0, tgt1)
    semh = (semh0, semh1)
    semx = (semx0, semx1)

    def zero_body(i, carry):
        hist_v[pl.ds(i * 16, 16)] = zeros_f
        return carry

    lax.fori_loop(0, (16 * _HIST_PAD) // 16, zero_body, 0)

    def start_h(ci):
        buf = ci % 2
        base = (wid * _NCHUNKS + ci) * _CHUNK
        a = pltpu.async_copy(
            out_hbm.at[pl.ds(base * _NUM_CLASSES, _CW)],
            shf.at[buf, s_id], semh[buf])
        b = pltpu.async_copy(
            tgt_hbm.at[pl.ds(base, _CHUNK)], sht.at[buf, s_id], semh[buf])
        return a, b

    def start_x(ci):
        buf = ci % 2
        a = pltpu.async_copy(shf.at[buf, s_id], chunks[buf], semx[buf])
        b = pltpu.async_copy(sht.at[buf, s_id], tgts[buf], semx[buf])
        return a, b

    pend_h = {0: start_h(0)}
    pend_h[1] = start_h(1)
    for d in pend_h.pop(0):
        d.wait()
    pend_x = {0: start_x(0)}

    for ci in range(_NCHUNKS):
        buf = ci % 2
        for d in pend_x.pop(ci):
            d.wait()
        if ci + 1 < _NCHUNKS:
            for d in pend_h.pop(ci + 1):
                d.wait()
            pend_x[ci + 1] = start_x(ci + 1)
        if ci + 2 < _NCHUNKS:
            pend_h[ci + 2] = start_h(ci + 2)
        chunk_v, tgt_v = chunks[buf], tgts[buf]

        def group_body(g, inner):
            row_base = (g * 16 + lanes) * _NUM_CLASSES
            arg = _argmax16(chunk_v, row_base)
            t = tgt_v[pl.ds(g * 16, 16)]
            flat = lanes * _HIST_PAD + t * _NUM_CLASSES + arg
            plsc.addupdate_scatter(hist_v, [flat], ones_f)
            return inner

        lax.fori_loop(0, _GROUPS, group_body, 0)

    def red_body(b, carry):
        acc = zeros_f
        for l in range(16):
            acc = acc + hist_v[pl.ds(l * _HIST_PAD + b * 16, 16)]
        res_v[pl.ds(b * 16, 16)] = acc
        return carry

    lax.fori_loop(0, _HIST_PAD // 16, red_body, 0)
    pltpu.sync_copy(res_v, part_hbm.at[wid])


@jax.jit
def kernel(output, target):
    mesh = plsc.VectorSubcoreMesh(core_axis_name="c", subcore_axis_name="s")
    run = functools.partial(
        pl.kernel,
        mesh=mesh,
        out_type=jax.ShapeDtypeStruct((_NW, _HIST_PAD), jnp.float32),
        scratch_types=[
            pltpu.VMEM((_CW,), jnp.float32),
            pltpu.VMEM((_CW,), jnp.float32),
            pltpu.VMEM((_CHUNK,), jnp.int32),
            pltpu.VMEM((_CHUNK,), jnp.int32),
            pltpu.VMEM((16 * _HIST_PAD,), jnp.float32),
            pltpu.VMEM((_HIST_PAD,), jnp.float32),
            pltpu.VMEM_SHARED((2, 16, _CW), jnp.float32),
            pltpu.VMEM_SHARED((2, 16, _CHUNK), jnp.int32),
            pltpu.SemaphoreType.DMA,
            pltpu.SemaphoreType.DMA,
            pltpu.SemaphoreType.DMA,
            pltpu.SemaphoreType.DMA,
        ],
        compiler_params=pltpu.CompilerParams(needs_layout_passes=False),
    )(_cm_body)
    parts = run(output.reshape(-1), target)
    cm = parts.sum(axis=0)[: _NUM_CLASSES * _NUM_CLASSES]
    return cm.reshape(_NUM_CLASSES, _NUM_CLASSES)


# speedup vs baseline: 1.0039x; 1.0039x over previous
import functools

import jax
import jax.numpy as jnp
from jax import lax
from jax.experimental import pallas as pl
from jax.experimental.pallas import tpu as pltpu
from jax.experimental.pallas import tpu_sc as plsc

_NUM_CLASSES = 21
_N = 1048576
_NW = 32
_ROWS_PER_TILE = _N // _NW
_CHUNK = 2048
_CW = _CHUNK * _NUM_CLASSES
_CB = _CW // 128
_TB = _CHUNK // 128
_NCHUNKS = _ROWS_PER_TILE // _CHUNK
_GROUPS = _CHUNK // 16
_HIST_PAD = 448


def _argmax16(chunk_ref, row_base):
    nodes = []
    for c in range(_NUM_CLASSES):
        f = row_base + c
        v = plsc.load_gather(
            chunk_ref, [jnp.right_shift(f, 7), jnp.bitwise_and(f, 127)])
        nodes.append((v, jnp.full((16,), c, jnp.int32)))
    while len(nodes) > 1:
        nxt = []
        for i in range(0, len(nodes) - 1, 2):
            (va, ia), (vb, ib) = nodes[i], nodes[i + 1]
            m = vb > va
            nxt.append((jnp.where(m, vb, va), jnp.where(m, ib, ia)))
        if len(nodes) % 2:
            nxt.append(nodes[-1])
        nodes = nxt
    return nodes[0][1]


def _cm_body(out_hbm, tgt_hbm, part_hbm, chunk0, chunk1, tgt0, tgt1,
             idx0, idx1, hist_v, res_v, sem0, sem1):
    s_id = lax.axis_index("s")
    wid = lax.axis_index("c") * 16 + s_id
    lanes = lax.broadcasted_iota(jnp.int32, (16,), 0)
    zeros_f = jnp.zeros((16,), jnp.float32)
    ones_f = jnp.ones((16,), jnp.float32)
    chunks = (chunk0, chunk1)
    tgts = (tgt0, tgt1)
    idxs = (idx0, idx1)
    sems = (sem0, sem1)

    def zero_body(i, carry):
        hist_v[pl.ds(i * 16, 16)] = zeros_f
        return carry

    lax.fori_loop(0, (16 * _HIST_PAD) // 16, zero_body, 0)

    def start(ci, buf):
        blk = (wid * _NCHUNKS + ci) * _CB
        tblk = (wid * _NCHUNKS + ci) * _TB
        for j in range(_CB // 16):
            idxs[buf][j // 7, pl.ds((j % 7) * 16, 16)] = blk + j * 16 + lanes
        copies = []
        for j in range(3):
            copies.append(pltpu.async_copy(
                out_hbm.at[idxs[buf].at[j]],
                chunks[buf].at[pl.ds(j * 112, 112)], sems[buf]))
        copies.append(pltpu.async_copy(
            tgt_hbm.at[tblk + lanes], tgts[buf], sems[buf]))
        return copies

    pending = start(0, 0)
    for ci in range(_NCHUNKS):
        buf = ci % 2
        for d in pending:
            d.wait()
        if ci + 1 < _NCHUNKS:
            pending = start(ci + 1, buf ^ 1)
        chunk_v, tgt_v = chunks[buf], tgts[buf]

        def group_body(g, inner):
            row_base = (g * 16 + lanes) * _NUM_CLASSES
            arg = _argmax16(chunk_v, row_base)
            tf = g * 16 + lanes
            t = plsc.load_gather(
                tgt_v, [jnp.right_shift(tf, 7), jnp.bitwise_and(tf, 127)])
            flat = lanes * _HIST_PAD + t * _NUM_CLASSES + arg
            plsc.addupdate_scatter(hist_v, [flat], ones_f)
            return inner

        lax.fori_loop(0, _GROUPS, group_body, 0)

    def red_body(b, carry):
        acc = zeros_f
        for l in range(16):
            acc = acc + hist_v[pl.ds(l * _HIST_PAD + b * 16, 16)]
        res_v[pl.ds(b * 16, 16)] = acc
        return carry

    lax.fori_loop(0, _HIST_PAD // 16, red_body, 0)
    pltpu.sync_copy(res_v, part_hbm.at[wid])


@jax.jit
def kernel(output, target):
    mesh = plsc.VectorSubcoreMesh(core_axis_name="c", subcore_axis_name="s")
    run = functools.partial(
        pl.kernel,
        mesh=mesh,
        out_type=jax.ShapeDtypeStruct((_NW, _HIST_PAD), jnp.float32),
        scratch_types=[
            pltpu.VMEM((_CB, 128), jnp.float32),
            pltpu.VMEM((_CB, 128), jnp.float32),
            pltpu.VMEM((_TB, 128), jnp.int32),
            pltpu.VMEM((_TB, 128), jnp.int32),
            pltpu.VMEM((3, 112), jnp.int32),
            pltpu.VMEM((3, 112), jnp.int32),
            pltpu.VMEM((16 * _HIST_PAD,), jnp.float32),
            pltpu.VMEM((_HIST_PAD,), jnp.float32),
            pltpu.SemaphoreType.DMA,
            pltpu.SemaphoreType.DMA,
        ],
        compiler_params=pltpu.CompilerParams(needs_layout_passes=False),
    )(_cm_body)
    parts = run(output.reshape(_N * _NUM_CLASSES // 128, 128),
                target.reshape(_N // 128, 128))
    cm = parts.sum(axis=0)[: _NUM_CLASSES * _NUM_CLASSES]
    return cm.reshape(_NUM_CLASSES, _NUM_CLASSES)


# speedup vs baseline: 7.9115x; 7.8805x over previous
import functools

import jax
import jax.numpy as jnp
from jax import lax
from jax.experimental import pallas as pl
from jax.experimental.pallas import tpu as pltpu
from jax.experimental.pallas import tpu_sc as plsc

_NUM_CLASSES = 21
_N = 1048576
_NW = 32
_ROWS_PER_TILE = _N // _NW
_CHUNK = 2048
_NCHUNKS = _ROWS_PER_TILE // _CHUNK
_GROUPS = _CHUNK // 16
_HIST_PAD = 448


def _argmax16(chunk_ref, r0):
    nodes = []
    for c in range(_NUM_CLASSES):
        v = chunk_ref[c, pl.ds(r0, 16)]
        nodes.append((v, jnp.full((16,), c, jnp.int32)))
    while len(nodes) > 1:
        nxt = []
        for i in range(0, len(nodes) - 1, 2):
            (va, ia), (vb, ib) = nodes[i], nodes[i + 1]
            m = vb > va
            nxt.append((jnp.where(m, vb, va), jnp.where(m, ib, ia)))
        if len(nodes) % 2:
            nxt.append(nodes[-1])
        nodes = nxt
    return nodes[0][1]


def _cm_body(out_hbm, tgt_hbm, part_hbm, chunk0, chunk1, tgt0, tgt1,
             hist_v, res_v, sem0, sem1):
    s_id = lax.axis_index("s")
    wid = lax.axis_index("c") * 16 + s_id
    lanes = lax.broadcasted_iota(jnp.int32, (16,), 0)
    zeros_f = jnp.zeros((16,), jnp.float32)
    ones_f = jnp.ones((16,), jnp.float32)
    chunks = (chunk0, chunk1)
    tgts = (tgt0, tgt1)
    sems = (sem0, sem1)

    def zero_body(i, carry):
        hist_v[pl.ds(i * 16, 16)] = zeros_f
        return carry

    lax.fori_loop(0, (16 * _HIST_PAD) // 16, zero_body, 0)

    def start(ci, buf):
        base = (wid * _NCHUNKS + ci) * _CHUNK
        a = pltpu.async_copy(
            out_hbm.at[:, pl.ds(base, _CHUNK)], chunks[buf], sems[buf])
        b = pltpu.async_copy(
            tgt_hbm.at[pl.ds(base, _CHUNK)], tgts[buf], sems[buf])
        return a, b

    pending = start(0, 0)
    for ci in range(_NCHUNKS):
        buf = ci % 2
        for d in pending:
            d.wait()
        if ci + 1 < _NCHUNKS:
            pending = start(ci + 1, buf ^ 1)
        chunk_v, tgt_v = chunks[buf], tgts[buf]

        def group_body(g, inner):
            r0 = g * 16
            arg = _argmax16(chunk_v, r0)
            t = tgt_v[pl.ds(r0, 16)]
            flat = lanes * _HIST_PAD + t * _NUM_CLASSES + arg
            plsc.addupdate_scatter(hist_v, [flat], ones_f)
            return inner

        lax.fori_loop(0, _GROUPS, group_body, 0)

    def red_body(b, carry):
        acc = zeros_f
        for l in range(16):
            acc = acc + hist_v[pl.ds(l * _HIST_PAD + b * 16, 16)]
        res_v[pl.ds(b * 16, 16)] = acc
        return carry

    lax.fori_loop(0, _HIST_PAD // 16, red_body, 0)
    pltpu.sync_copy(res_v, part_hbm.at[pl.ds(wid * _HIST_PAD, _HIST_PAD)])


@jax.jit
def kernel(output, target):
    mesh = plsc.VectorSubcoreMesh(core_axis_name="c", subcore_axis_name="s")
    run = functools.partial(
        pl.kernel,
        mesh=mesh,
        out_type=jax.ShapeDtypeStruct((_NW * _HIST_PAD,), jnp.float32),
        scratch_types=[
            pltpu.VMEM((_NUM_CLASSES, _CHUNK), jnp.float32),
            pltpu.VMEM((_NUM_CLASSES, _CHUNK), jnp.float32),
            pltpu.VMEM((_CHUNK,), jnp.int32),
            pltpu.VMEM((_CHUNK,), jnp.int32),
            pltpu.VMEM((16 * _HIST_PAD,), jnp.float32),
            pltpu.VMEM((_HIST_PAD,), jnp.float32),
            pltpu.SemaphoreType.DMA,
            pltpu.SemaphoreType.DMA,
        ],
        compiler_params=pltpu.CompilerParams(
            needs_layout_passes=False, use_tc_tiling_on_sc=True),
    )(_cm_body)
    parts = run(output.T, target)
    cm = parts.reshape(_NW, _HIST_PAD).sum(axis=0)[: _NUM_CLASSES * _NUM_CLASSES]
    return cm.reshape(_NUM_CLASSES, _NUM_CLASSES)


# speedup vs baseline: 9.7392x; 1.2310x over previous
import functools

import jax
import jax.numpy as jnp
from jax import lax
from jax.experimental import pallas as pl
from jax.experimental.pallas import tpu as pltpu
from jax.experimental.pallas import tpu_sc as plsc

_NUM_CLASSES = 21
_N = 1048576
_NW = 32
_CHUNK = 2048
_NCHUNKS = 10
_N_SC = _NW * _CHUNK * _NCHUNKS
_GROUPS = _CHUNK // 16
_HIST_PAD = 448
_BT = 16384
_TC_BLOCKS = (_N - _N_SC) // _BT
_TC_OFF = _N_SC // _BT


def _argmax16(chunk_ref, r0):
    nodes = []
    for c in range(_NUM_CLASSES):
        v = chunk_ref[c, pl.ds(r0, 16)]
        nodes.append((v, jnp.full((16,), c, jnp.int32)))
    while len(nodes) > 1:
        nxt = []
        for i in range(0, len(nodes) - 1, 2):
            (va, ia), (vb, ib) = nodes[i], nodes[i + 1]
            m = vb > va
            nxt.append((jnp.where(m, vb, va), jnp.where(m, ib, ia)))
        if len(nodes) % 2:
            nxt.append(nodes[-1])
        nodes = nxt
    return nodes[0][1]


def _cm_body(out_hbm, tgt_hbm, part_hbm, chunk0, chunk1, tgt0, tgt1,
             hist_v, res_v, sem0, sem1):
    s_id = lax.axis_index("s")
    wid = lax.axis_index("c") * 16 + s_id
    lanes = lax.broadcasted_iota(jnp.int32, (16,), 0)
    zeros_f = jnp.zeros((16,), jnp.float32)
    ones_f = jnp.ones((16,), jnp.float32)
    chunks = (chunk0, chunk1)
    tgts = (tgt0, tgt1)
    sems = (sem0, sem1)

    def zero_body(i, carry):
        hist_v[pl.ds(i * 16, 16)] = zeros_f
        return carry

    lax.fori_loop(0, (16 * _HIST_PAD) // 16, zero_body, 0)

    def start(ci, buf):
        base = (wid * _NCHUNKS + ci) * _CHUNK
        a = pltpu.async_copy(
            out_hbm.at[:, pl.ds(base, _CHUNK)], chunks[buf], sems[buf])
        b = pltpu.async_copy(
            tgt_hbm.at[pl.ds(base, _CHUNK)], tgts[buf], sems[buf])
        return a, b

    pending = start(0, 0)
    for ci in range(_NCHUNKS):
        buf = ci % 2
        for d in pending:
            d.wait()
        if ci + 1 < _NCHUNKS:
            pending = start(ci + 1, buf ^ 1)
        chunk_v, tgt_v = chunks[buf], tgts[buf]

        def group_body(g, inner):
            r0 = g * 16
            arg = _argmax16(chunk_v, r0)
            t = tgt_v[pl.ds(r0, 16)]
            flat = lanes * _HIST_PAD + t * _NUM_CLASSES + arg
            plsc.addupdate_scatter(hist_v, [flat], ones_f)
            return inner

        lax.fori_loop(0, _GROUPS, group_body, 0)

    def red_body(b, carry):
        acc = zeros_f
        for l in range(16):
            acc = acc + hist_v[pl.ds(l * _HIST_PAD + b * 16, 16)]
        res_v[pl.ds(b * 16, 16)] = acc
        return carry

    lax.fori_loop(0, _HIST_PAD // 16, red_body, 0)
    pltpu.sync_copy(res_v, part_hbm.at[pl.ds(wid * _HIST_PAD, _HIST_PAD)])


def _tc_body(x_ref, t_ref, o_ref):
    x = x_ref[...]
    am = jnp.argmax(x, axis=0).astype(jnp.int32)
    cls = lax.broadcasted_iota(jnp.int32, (_NUM_CLASSES, _BT), 0)
    ohp = (cls == am[None, :]).astype(jnp.float32)
    oht = (cls == t_ref[...]).astype(jnp.float32)
    cm = lax.dot_general(oht, ohp, (((1,), (1,)), ((), ())),
                         preferred_element_type=jnp.float32)

    @pl.when(pl.program_id(0) == 0)
    def _():
        o_ref[...] = jnp.zeros_like(o_ref)

    o_ref[...] += cm


@jax.jit
def kernel(output, target):
    out_t = output.T
    mesh = plsc.VectorSubcoreMesh(core_axis_name="c", subcore_axis_name="s")
    run = functools.partial(
        pl.kernel,
        mesh=mesh,
        out_type=jax.ShapeDtypeStruct((_NW * _HIST_PAD,), jnp.float32),
        scratch_types=[
            pltpu.VMEM((_NUM_CLASSES, _CHUNK), jnp.float32),
            pltpu.VMEM((_NUM_CLASSES, _CHUNK), jnp.float32),
            pltpu.VMEM((_CHUNK,), jnp.int32),
            pltpu.VMEM((_CHUNK,), jnp.int32),
            pltpu.VMEM((16 * _HIST_PAD,), jnp.float32),
            pltpu.VMEM((_HIST_PAD,), jnp.float32),
            pltpu.SemaphoreType.DMA,
            pltpu.SemaphoreType.DMA,
        ],
        compiler_params=pltpu.CompilerParams(
            needs_layout_passes=False, use_tc_tiling_on_sc=True),
    )(_cm_body)
    parts = run(out_t, target)

    tc_cm = pl.pallas_call(
        _tc_body,
        grid=(_TC_BLOCKS,),
        in_specs=[
            pl.BlockSpec((_NUM_CLASSES, _BT), lambda i: (0, _TC_OFF + i)),
            pl.BlockSpec((1, _BT), lambda i: (0, _TC_OFF + i)),
        ],
        out_specs=pl.BlockSpec((_NUM_CLASSES, _NUM_CLASSES), lambda i: (0, 0)),
        out_shape=jax.ShapeDtypeStruct((_NUM_CLASSES, _NUM_CLASSES),
                                       jnp.float32),
    )(out_t, target.reshape(1, _N))

    sc_cm = parts.reshape(_NW, _HIST_PAD).sum(axis=0)[
        : _NUM_CLASSES * _NUM_CLASSES].reshape(_NUM_CLASSES, _NUM_CLASSES)
    return sc_cm + tc_cm
